# baseline (speedup 1.0000x reference)
import jax
import jax.numpy as jnp
from jax.experimental import pallas as pl

_N = 300
_SEQ = 128
_HEADS = 2
_LOG2E = 1.4426950408889634


def _gat_dense_kernel(x_ref, w_ref, asrc_ref, adst_ref, bias_ref, out_ref):
    for b in range(x_ref.shape[0]):
        xb = x_ref[b]
        hT = jax.lax.dot_general(w_ref[...], xb, (((0,), (0,)), ((), ())),
                                 preferred_element_type=jnp.float32)
        acc = jnp.transpose(bias_ref[...])
        for h in range(_HEADS):
            hhT = hT[h * _SEQ:(h + 1) * _SEQ, :]
            a_src = _LOG2E * jnp.dot(asrc_ref[h:h + 1, :], hhT,
                                     preferred_element_type=jnp.float32)
            a_dst = _LOG2E * jnp.dot(adst_ref[h:h + 1, :], hhT,
                                     preferred_element_type=jnp.float32)
            logits = jnp.transpose(a_src) + a_dst
            logits = jnp.maximum(logits, 0.2 * logits)
            m = jnp.max(a_src) + a_dst
            m = jnp.maximum(m, 0.2 * m)
            e = jnp.exp2(logits - m)
            s = jnp.sum(e, axis=0, keepdims=True)
            o = jnp.dot(hhT, e, preferred_element_type=jnp.float32)
            acc = acc + o * ((1.0 / _HEADS) / (s + 1e-16))
        out_ref[b] = acc


def kernel(x, W, att_src, att_dst, bias, edge_index):
    del edge_index
    B = x.shape[0]
    bias2 = bias.reshape(1, _SEQ)
    return pl.pallas_call(
        _gat_dense_kernel,
        in_specs=[
            pl.BlockSpec((B, _SEQ, _N), lambda: (0, 0, 0)),
            pl.BlockSpec((_SEQ, _HEADS * _SEQ), lambda: (0, 0)),
            pl.BlockSpec((_HEADS, _SEQ), lambda: (0, 0)),
            pl.BlockSpec((_HEADS, _SEQ), lambda: (0, 0)),
            pl.BlockSpec((1, _SEQ), lambda: (0, 0)),
        ],
        out_specs=pl.BlockSpec((B, _SEQ, _N), lambda: (0, 0, 0)),
        out_shape=jax.ShapeDtypeStruct((B, _SEQ, _N), jnp.float32),
    )(x, W, att_src, att_dst, bias2)
